# baseline (speedup 1.0000x reference)
import functools

import jax
import jax.numpy as jnp
from jax import lax
from jax.experimental import pallas as pl
from jax.experimental.pallas import tpu as pltpu
from jax.experimental.pallas import tpu_sc as plsc

F32 = jnp.float32

N = 10000
G = 128
HID = 512
NPAD = 10240
NSUB = 16
NODES_PER_TILE = NPAD // NSUB
EPAD = 655360
ROWS = EPAD // 128
ROWS_PER_TILE = ROWS // NSUB
CHUNK_ROWS = 8

@functools.cache
def _mesh():
    return plsc.VectorSubcoreMesh(core_axis_name="c", subcore_axis_name="s")


def _deg_body(srcp, dstp, out, idx_v, ones_v, zero_v, deg_sp):
    core = lax.axis_index("c")
    sub = lax.axis_index("s")
    for i in range(8):
        ones_v[0, pl.ds(i * 16, 16)] = jnp.ones((16,), F32)
    for i in range(NODES_PER_TILE // 16):
        zero_v[pl.ds(i * 16, 16)] = jnp.zeros((16,), F32)
    pltpu.sync_copy(zero_v, deg_sp.at[pl.ds(sub * NODES_PER_TILE, NODES_PER_TILE)])
    plsc.subcore_barrier()

    def run(idx_hbm):
        base = sub * ROWS_PER_TILE

        def chunk(ci, carry):
            row = base + ci * CHUNK_ROWS
            pltpu.sync_copy(idx_hbm.at[pl.ds(row, CHUNK_ROWS)], idx_v)
            for r in range(CHUNK_ROWS):
                pltpu.sync_copy(ones_v.at[0], deg_sp.at[idx_v.at[r]], add=True)
            return carry

        lax.fori_loop(0, ROWS_PER_TILE // CHUNK_ROWS, chunk, 0)

    @pl.when(core == 0)
    def _():
        run(dstp)

    @pl.when(core == 1)
    def _():
        run(srcp)

    plsc.subcore_barrier()
    off = core * NPAD + sub * NODES_PER_TILE
    pltpu.sync_copy(deg_sp.at[pl.ds(sub * NODES_PER_TILE, NODES_PER_TILE)],
                    out.at[pl.ds(off, NODES_PER_TILE)])


@functools.cache
def _deg_kernel():
    return pl.kernel(
        _deg_body,
        out_type=jax.ShapeDtypeStruct((2 * NPAD,), F32),
        mesh=_mesh(),
        scratch_types=[
            pltpu.VMEM((CHUNK_ROWS, 128), jnp.int32),
            pltpu.VMEM((1, 128), F32),
            pltpu.VMEM((NODES_PER_TILE,), F32),
            pltpu.VMEM_SHARED((NPAD,), F32),
        ],
    )


def _rsqrt16(x):
    xh = x * 0.5
    i = plsc.bitcast(x, jnp.int32)
    i = 0x5F3759DF - lax.shift_right_logical(i, 1)
    y = plsc.bitcast(i, F32)
    for _ in range(3):
        y = y * (1.5 - xh * y * y)
    return y


def _tsum_body(deg_hbm, srcp, dstp, out, v_v, dloc_v, sidx_v, didx_v,
               contrib_v, zero_v, t_sp, v_sp):
    core = lax.axis_index("c")
    sub = lax.axis_index("s")
    for i in range(NODES_PER_TILE // 16):
        zero_v[pl.ds(i * 16, 16)] = jnp.zeros((16,), F32)
    pltpu.sync_copy(zero_v, t_sp.at[pl.ds(sub * NODES_PER_TILE, NODES_PER_TILE)])
    nbase = sub * NODES_PER_TILE
    pltpu.sync_copy(deg_hbm.at[pl.ds(nbase, NODES_PER_TILE)],
                    dloc_v.at[pl.ds(0, NODES_PER_TILE)])
    pltpu.sync_copy(deg_hbm.at[pl.ds(NPAD + nbase, NODES_PER_TILE)],
                    dloc_v.at[pl.ds(NODES_PER_TILE, NODES_PER_TILE)])
    for i in range(NODES_PER_TILE // 16):
        di = dloc_v[pl.ds(i * 16, 16)]
        do = dloc_v[pl.ds(NODES_PER_TILE + i * 16, 16)]
        dloc_v[pl.ds(i * 16, 16)] = di * _rsqrt16(jnp.maximum(do, 1.0))
    pltpu.sync_copy(dloc_v.at[pl.ds(0, NODES_PER_TILE)],
                    v_sp.at[pl.ds(nbase, NODES_PER_TILE)])
    plsc.subcore_barrier()
    pltpu.sync_copy(v_sp, v_v)

    rows_here = ROWS // (2 * NSUB)
    base = (core * NSUB + sub) * rows_here

    def chunk(ci, carry):
        row = base + ci * CHUNK_ROWS
        pltpu.sync_copy(srcp.at[pl.ds(row, CHUNK_ROWS)], sidx_v)
        pltpu.sync_copy(dstp.at[pl.ds(row, CHUNK_ROWS)], didx_v)
        for r in range(CHUNK_ROWS):
            for c in range(8):
                s16 = sidx_v[r, pl.ds(c * 16, 16)]
                contrib_v[r, pl.ds(c * 16, 16)] = plsc.load_gather(v_v, [s16])
        for r in range(CHUNK_ROWS):
            pltpu.sync_copy(contrib_v.at[r], t_sp.at[didx_v.at[r]], add=True)
        return carry

    lax.fori_loop(0, rows_here // CHUNK_ROWS, chunk, 0)

    plsc.subcore_barrier()
    off = core * NPAD + sub * NODES_PER_TILE
    pltpu.sync_copy(t_sp.at[pl.ds(sub * NODES_PER_TILE, NODES_PER_TILE)],
                    out.at[pl.ds(off, NODES_PER_TILE)])


@functools.cache
def _tsum_kernel():
    return pl.kernel(
        _tsum_body,
        out_type=jax.ShapeDtypeStruct((2 * NPAD,), F32),
        mesh=_mesh(),
        compiler_params=pltpu.CompilerParams(needs_layout_passes=False),
        scratch_types=[
            pltpu.VMEM((NPAD,), F32),
            pltpu.VMEM((2 * NODES_PER_TILE,), F32),
            pltpu.VMEM((CHUNK_ROWS, 128), jnp.int32),
            pltpu.VMEM((CHUNK_ROWS, 128), jnp.int32),
            pltpu.VMEM((CHUNK_ROWS, 128), F32),
            pltpu.VMEM((NODES_PER_TILE,), F32),
            pltpu.VMEM_SHARED((NPAD,), F32),
            pltpu.VMEM_SHARED((NPAD,), F32),
        ],
    )


ACHUNK = 32
NCHUNK = ROWS_PER_TILE // ACHUNK
IBLK = 2 * ACHUNK


def _agg_body(y_hbm, edg, out, ib0, ib1, r0, r1, acc_sp,
              g0, g1, is0, is1, s0, s1):
    core = lax.axis_index("c")
    sub = lax.axis_index("s")
    rows = (r0, r1)
    gs = (g0, g1)
    ss = (s0, s1)
    ibs = (ib0, ib1)
    iss = (is0, is1)

    def sdrain(b):
        pltpu.make_async_copy(y_hbm.at[pl.ds(0, 128)], rows[b], ss[b]).wait()

    for j in range(2):
        p = core * 2 + j

        def zfill(i, carry):
            for c in range(8):
                r0[i, pl.ds(c * 16, 16)] = jnp.zeros((16,), F32)
            return carry

        lax.fori_loop(0, 128, zfill, 0)
        for k in range(NODES_PER_TILE // 128):
            pltpu.sync_copy(r0, acc_sp.at[pl.ds(sub * NODES_PER_TILE + k * 128, 128)])
        plsc.subcore_barrier()

        base_blk = (p * NSUB + sub) * NCHUNK
        pltpu.async_copy(edg.at[pl.ds(base_blk * IBLK, IBLK)], ib0, is0)
        pltpu.async_copy(y_hbm.at[pl.ds(0, 128)], r0, s0)
        pltpu.async_copy(y_hbm.at[pl.ds(0, 128)], r1, s1)

        def super_chunk(si, carry):
            for half in range(2):
                ci = si * 2 + half
                ib, isem = ibs[half], iss[half]
                nib, nisem = ibs[1 - half], iss[1 - half]
                pltpu.make_async_copy(edg.at[pl.ds(0, IBLK)], ib, isem).wait()
                nblk = jnp.minimum(base_blk + ci + 1, base_blk + NCHUNK - 1)
                pltpu.async_copy(edg.at[pl.ds(nblk * IBLK, IBLK)], nib, nisem)
                cps = {}
                sdrain(0)
                cps[0] = pltpu.async_copy(y_hbm.at[ib.at[0]], rows[0], gs[0])
                for r in range(ACHUNK):
                    b = r % 2
                    cps[r].wait()
                    if r + 1 < ACHUNK:
                        b1 = (r + 1) % 2
                        sdrain(b1)
                        cps[r + 1] = pltpu.async_copy(
                            y_hbm.at[ib.at[r + 1]], rows[b1], gs[b1])
                    pltpu.async_copy(rows[b], acc_sp.at[ib.at[ACHUNK + r]],
                                     ss[b], add=True)
            return carry

        lax.fori_loop(0, NCHUNK // 2, super_chunk, 0)
        pltpu.make_async_copy(edg.at[pl.ds(0, IBLK)], ib0, is0).wait()
        sdrain(0)
        sdrain(1)

        plsc.subcore_barrier()
        pltpu.sync_copy(acc_sp.at[pl.ds(sub * NODES_PER_TILE, NODES_PER_TILE)],
                        out.at[pl.ds(p * NPAD + sub * NODES_PER_TILE, NODES_PER_TILE)])


@functools.cache
def _agg_kernel():
    return pl.kernel(
        _agg_body,
        out_type=jax.ShapeDtypeStruct((4 * NPAD, 128), F32),
        mesh=_mesh(),
        scratch_types=[
            pltpu.VMEM((IBLK, 128), jnp.int32),
            pltpu.VMEM((IBLK, 128), jnp.int32),
            pltpu.VMEM((128, 128), F32),
            pltpu.VMEM((128, 128), F32),
            pltpu.VMEM_SHARED((NPAD, 128), F32),
            pltpu.SemaphoreType.DMA,
            pltpu.SemaphoreType.DMA,
            pltpu.SemaphoreType.DMA,
            pltpu.SemaphoreType.DMA,
            pltpu.SemaphoreType.DMA,
            pltpu.SemaphoreType.DMA,
        ],
    )


def _h1y_body(t2_ref, deg2_ref, w1_ref, b1_ref, w2_ref, y_ref):
    t = t2_ref[0] + t2_ref[1]
    nd = lax.rsqrt(jnp.maximum(deg2_ref[0], 1.0))
    ns = lax.rsqrt(jnp.maximum(deg2_ref[1], 1.0))
    u = t * nd
    h1 = jnp.maximum(u * w1_ref[...] + b1_ref[...], 0.0)
    h1s = h1 * ns
    y_ref[...] = jnp.dot(h1s, w2_ref[...], preferred_element_type=F32)


def _tc_h1y(t2, deg2, w1, b1, w2):
    nr = NPAD // 1024
    return pl.pallas_call(
        _h1y_body,
        grid=(nr, 4),
        in_specs=[
            pl.BlockSpec((2, 1024, 1), lambda r, p: (0, r, 0)),
            pl.BlockSpec((2, 1024, 1), lambda r, p: (0, r, 0)),
            pl.BlockSpec((1, HID), lambda r, p: (0, 0)),
            pl.BlockSpec((1, HID), lambda r, p: (0, 0)),
            pl.BlockSpec((HID, 128), lambda r, p: (0, p)),
        ],
        out_specs=pl.BlockSpec((1024, 128), lambda r, p: (p * nr + r, 0)),
        out_shape=jax.ShapeDtypeStruct((4 * NPAD, 128), F32),
    )(t2, deg2, w1, b1, w2)


def _pool_mlp_body(agg_ref, din_ref, b2_ref, gid_ref,
                   w1, b1, w2, b2, w3, b3, w4, b4, w5, b5,
                   out_ref, pooled_ref, cnt_ref):
    r = pl.program_id(0)
    nr = pl.num_programs(0)
    agg = jnp.concatenate([agg_ref[i] for i in range(4)], axis=1)
    nd = lax.rsqrt(jnp.maximum(din_ref[...], 1.0))
    h2 = jnp.maximum(agg * nd + b2_ref[...], 0.0)
    gid = gid_ref[...]
    pt = (lax.broadcasted_iota(jnp.int32, (G, 1024), 0)
          == gid).astype(F32)
    pp = jnp.dot(pt, h2, preferred_element_type=F32)
    pc = jnp.dot(pt, jnp.ones((1024, 1), F32), preferred_element_type=F32)

    @pl.when(r == 0)
    def _():
        pooled_ref[...] = pp
        cnt_ref[...] = pc

    @pl.when(r > 0)
    def _():
        pooled_ref[...] += pp
        cnt_ref[...] += pc

    @pl.when(r == nr - 1)
    def _():
        hg = pooled_ref[...] / jnp.maximum(cnt_ref[...], 1.0)
        x = jnp.maximum(jnp.dot(hg, w1[...], preferred_element_type=F32) + b1[...], 0.0)
        x = jnp.maximum(jnp.dot(x, w2[...], preferred_element_type=F32) + b2[...], 0.0)
        x = jnp.maximum(jnp.dot(x, w3[...], preferred_element_type=F32) + b3[...], 0.0)
        x = jnp.maximum(jnp.dot(x, w4[...], preferred_element_type=F32) + b4[...], 0.0)
        logits = jnp.dot(x, w5[...], preferred_element_type=F32) + b5[...]
        m = jnp.max(logits, axis=-1, keepdims=True)
        e = jnp.exp(logits - m)
        out_ref[...] = e / jnp.sum(e, axis=-1, keepdims=True)


def _tc_pool_mlp(agg4, din, b2, gid_row,
                 wc1, bc1, wc2, bc2, wc3, bc3, wc4, bc4, wc5, bc5):
    nr = NPAD // 1024
    full = lambda arr: pl.BlockSpec(arr.shape, lambda r: (0,) * arr.ndim)
    return pl.pallas_call(
        _pool_mlp_body,
        grid=(nr,),
        in_specs=[
            pl.BlockSpec((4, 1024, 128), lambda r: (0, r, 0)),
            pl.BlockSpec((1024, 1), lambda r: (r, 0)),
            pl.BlockSpec((1, HID), lambda r: (0, 0)),
            pl.BlockSpec((1, 1024), lambda r: (0, r)),
            full(wc1), full(bc1), full(wc2), full(bc2), full(wc3),
            full(bc3), full(wc4), full(bc4), full(wc5), full(bc5),
        ],
        out_specs=pl.BlockSpec((G, 10), lambda r: (0, 0)),
        out_shape=jax.ShapeDtypeStruct((G, 10), F32),
        scratch_shapes=[
            pltpu.VMEM((G, HID), F32),
            pltpu.VMEM((G, 1), F32),
        ],
    )(agg4, din, b2, gid_row,
      wc1, bc1, wc2, bc2, wc3, bc3, wc4, bc4, wc5, bc5)


def kernel(edge_index, graph_ids, W1, b1, W2, b2,
           Wc1, bc1, Wc2, bc2, Wc3, bc3, Wc4, bc4, Wc5, bc5):
    E = edge_index.shape[1]
    src = edge_index[0]
    dst = edge_index[1]
    pad = N + (jnp.arange(EPAD - E, dtype=jnp.int32) % (NPAD - N))
    srcp = jnp.concatenate([src, pad]).reshape(ROWS, 128)
    dstp = jnp.concatenate([dst, pad]).reshape(ROWS, 128)
    srcoff = (srcp[None] + (jnp.arange(4, dtype=jnp.int32) * NPAD)[:, None, None]
              ).reshape(4, NSUB, NCHUNK, ACHUNK, 128)
    dstb = jnp.broadcast_to(dstp.reshape(1, NSUB, NCHUNK, ACHUNK, 128),
                            (4, NSUB, NCHUNK, ACHUNK, 128))
    edg = jnp.concatenate([srcoff, dstb], axis=3).reshape(4 * ROWS * 2, 128)

    deg = _deg_kernel()(srcp, dstp)
    deg2 = deg.reshape(2, NPAD, 1)

    t2 = _tsum_kernel()(deg, srcp, dstp).reshape(2, NPAD, 1)

    y = _tc_h1y(t2, deg2, W1, b1.reshape(1, HID), W2)

    agg = _agg_kernel()(y, edg).reshape(4, NPAD, 128)

    gid_row = jnp.concatenate(
        [graph_ids, jnp.full((NPAD - N,), 1000, jnp.int32)]).reshape(1, NPAD)
    return _tc_pool_mlp(agg, deg2[0], b2.reshape(1, HID), gid_row,
                        Wc1, bc1.reshape(1, -1), Wc2, bc2.reshape(1, -1),
                        Wc3, bc3.reshape(1, -1), Wc4, bc4.reshape(1, -1),
                        Wc5, bc5.reshape(1, -1))


# speedup vs baseline: 1.2109x; 1.2109x over previous
import functools

import jax
import jax.numpy as jnp
from jax import lax
from jax.experimental import pallas as pl
from jax.experimental.pallas import tpu as pltpu
from jax.experimental.pallas import tpu_sc as plsc

F32 = jnp.float32

N = 10000
G = 128
HID = 512
NPAD = 10240
NSUB = 16
NODES_PER_TILE = NPAD // NSUB
EPAD = 655360
ROWS = EPAD // 128
ROWS_PER_TILE = ROWS // NSUB
CHUNK_ROWS = 32

@functools.cache
def _mesh():
    return plsc.VectorSubcoreMesh(core_axis_name="c", subcore_axis_name="s")


def _deg_body(srcp, dstp, out, idx_v, ones_v, zero_v, deg_sp):
    core = lax.axis_index("c")
    sub = lax.axis_index("s")
    for i in range(8):
        ones_v[0, pl.ds(i * 16, 16)] = jnp.ones((16,), F32)
    for i in range(NODES_PER_TILE // 16):
        zero_v[pl.ds(i * 16, 16)] = jnp.zeros((16,), F32)
    pltpu.sync_copy(zero_v, deg_sp.at[pl.ds(sub * NODES_PER_TILE, NODES_PER_TILE)])
    plsc.subcore_barrier()

    def run(idx_hbm):
        base = sub * ROWS_PER_TILE

        def chunk(ci, carry):
            row = base + ci * CHUNK_ROWS
            pltpu.sync_copy(idx_hbm.at[pl.ds(row, CHUNK_ROWS)], idx_v)
            for r in range(CHUNK_ROWS):
                pltpu.sync_copy(ones_v.at[0], deg_sp.at[idx_v.at[r]], add=True)
            return carry

        lax.fori_loop(0, ROWS_PER_TILE // CHUNK_ROWS, chunk, 0)

    @pl.when(core == 0)
    def _():
        run(dstp)

    @pl.when(core == 1)
    def _():
        run(srcp)

    plsc.subcore_barrier()
    off = core * NPAD + sub * NODES_PER_TILE
    pltpu.sync_copy(deg_sp.at[pl.ds(sub * NODES_PER_TILE, NODES_PER_TILE)],
                    out.at[pl.ds(off, NODES_PER_TILE)])


@functools.cache
def _deg_kernel():
    return pl.kernel(
        _deg_body,
        out_type=jax.ShapeDtypeStruct((2 * NPAD,), F32),
        mesh=_mesh(),
        scratch_types=[
            pltpu.VMEM((CHUNK_ROWS, 128), jnp.int32),
            pltpu.VMEM((1, 128), F32),
            pltpu.VMEM((NODES_PER_TILE,), F32),
            pltpu.VMEM_SHARED((NPAD,), F32),
        ],
    )


def _rsqrt16(x):
    xh = x * 0.5
    i = plsc.bitcast(x, jnp.int32)
    i = 0x5F3759DF - lax.shift_right_logical(i, 1)
    y = plsc.bitcast(i, F32)
    for _ in range(3):
        y = y * (1.5 - xh * y * y)
    return y


def _tsum_body(deg_hbm, srcp, dstp, out, v_v, dloc_v, sidx_v, didx_v,
               contrib_v, zero_v, t_sp, v_sp):
    core = lax.axis_index("c")
    sub = lax.axis_index("s")
    for i in range(NODES_PER_TILE // 16):
        zero_v[pl.ds(i * 16, 16)] = jnp.zeros((16,), F32)
    pltpu.sync_copy(zero_v, t_sp.at[pl.ds(sub * NODES_PER_TILE, NODES_PER_TILE)])
    nbase = sub * NODES_PER_TILE
    pltpu.sync_copy(deg_hbm.at[pl.ds(nbase, NODES_PER_TILE)],
                    dloc_v.at[pl.ds(0, NODES_PER_TILE)])
    pltpu.sync_copy(deg_hbm.at[pl.ds(NPAD + nbase, NODES_PER_TILE)],
                    dloc_v.at[pl.ds(NODES_PER_TILE, NODES_PER_TILE)])
    for i in range(NODES_PER_TILE // 16):
        di = dloc_v[pl.ds(i * 16, 16)]
        do = dloc_v[pl.ds(NODES_PER_TILE + i * 16, 16)]
        dloc_v[pl.ds(i * 16, 16)] = di * _rsqrt16(jnp.maximum(do, 1.0))
    pltpu.sync_copy(dloc_v.at[pl.ds(0, NODES_PER_TILE)],
                    v_sp.at[pl.ds(nbase, NODES_PER_TILE)])
    plsc.subcore_barrier()
    pltpu.sync_copy(v_sp, v_v)

    rows_here = ROWS // (2 * NSUB)
    base = (core * NSUB + sub) * rows_here

    def chunk(ci, carry):
        row = base + ci * CHUNK_ROWS
        pltpu.sync_copy(srcp.at[pl.ds(row, CHUNK_ROWS)], sidx_v)
        pltpu.sync_copy(dstp.at[pl.ds(row, CHUNK_ROWS)], didx_v)
        for r in range(CHUNK_ROWS):
            for c in range(8):
                s16 = sidx_v[r, pl.ds(c * 16, 16)]
                contrib_v[r, pl.ds(c * 16, 16)] = plsc.load_gather(v_v, [s16])
        for r in range(CHUNK_ROWS):
            pltpu.sync_copy(contrib_v.at[r], t_sp.at[didx_v.at[r]], add=True)
        return carry

    lax.fori_loop(0, rows_here // CHUNK_ROWS, chunk, 0)

    plsc.subcore_barrier()
    off = core * NPAD + sub * NODES_PER_TILE
    pltpu.sync_copy(t_sp.at[pl.ds(sub * NODES_PER_TILE, NODES_PER_TILE)],
                    out.at[pl.ds(off, NODES_PER_TILE)])


@functools.cache
def _tsum_kernel():
    return pl.kernel(
        _tsum_body,
        out_type=jax.ShapeDtypeStruct((2 * NPAD,), F32),
        mesh=_mesh(),
        compiler_params=pltpu.CompilerParams(needs_layout_passes=False),
        scratch_types=[
            pltpu.VMEM((NPAD,), F32),
            pltpu.VMEM((2 * NODES_PER_TILE,), F32),
            pltpu.VMEM((CHUNK_ROWS, 128), jnp.int32),
            pltpu.VMEM((CHUNK_ROWS, 128), jnp.int32),
            pltpu.VMEM((CHUNK_ROWS, 128), F32),
            pltpu.VMEM((NODES_PER_TILE,), F32),
            pltpu.VMEM_SHARED((NPAD,), F32),
            pltpu.VMEM_SHARED((NPAD,), F32),
        ],
    )


ACHUNK = 32
NCHUNK = ROWS_PER_TILE // ACHUNK
IBLK = 2 * ACHUNK


def _agg_body(y_hbm, edg, out, ib0, ib1, r0, r1, acc_sp,
              g0, g1, is0, is1):
    core = lax.axis_index("c")
    sub = lax.axis_index("s")
    rows = (r0, r1)
    gs = (g0, g1)
    ibs = (ib0, ib1)
    iss = (is0, is1)

    for j in range(2):
        p = core * 2 + j

        def zfill(i, carry):
            for c in range(8):
                r0[i, pl.ds(c * 16, 16)] = jnp.zeros((16,), F32)
            return carry

        lax.fori_loop(0, 128, zfill, 0)
        for k in range(NODES_PER_TILE // 128):
            pltpu.sync_copy(r0, acc_sp.at[pl.ds(sub * NODES_PER_TILE + k * 128, 128)])
        plsc.subcore_barrier()

        base_blk = (p * NSUB + sub) * NCHUNK
        pltpu.async_copy(edg.at[pl.ds(base_blk * IBLK, IBLK)], ib0, is0)

        def super_chunk(si, carry):
            for half in range(2):
                ci = si * 2 + half
                ib, isem = ibs[half], iss[half]
                nib, nisem = ibs[1 - half], iss[1 - half]
                pltpu.make_async_copy(edg.at[pl.ds(0, IBLK)], ib, isem).wait()
                nblk = jnp.minimum(base_blk + ci + 1, base_blk + NCHUNK - 1)
                pltpu.async_copy(edg.at[pl.ds(nblk * IBLK, IBLK)], nib, nisem)
                cps = {}
                cps[0] = pltpu.async_copy(y_hbm.at[ib.at[0]], rows[0], gs[0])
                for r in range(ACHUNK):
                    if r + 1 < ACHUNK:
                        cps[r + 1] = pltpu.async_copy(
                            y_hbm.at[ib.at[r + 1]], rows[(r + 1) % 2], gs[(r + 1) % 2])
                    cps[r].wait()
                    pltpu.sync_copy(rows[r % 2], acc_sp.at[ib.at[ACHUNK + r]], add=True)
            return carry

        lax.fori_loop(0, NCHUNK // 2, super_chunk, 0)
        pltpu.make_async_copy(edg.at[pl.ds(0, IBLK)], ib0, is0).wait()

        plsc.subcore_barrier()
        pltpu.sync_copy(acc_sp.at[pl.ds(sub * NODES_PER_TILE, NODES_PER_TILE)],
                        out.at[pl.ds(p * NPAD + sub * NODES_PER_TILE, NODES_PER_TILE)])


@functools.cache
def _agg_kernel():
    return pl.kernel(
        _agg_body,
        out_type=jax.ShapeDtypeStruct((4 * NPAD, 128), F32),
        mesh=_mesh(),
        scratch_types=[
            pltpu.VMEM((IBLK, 128), jnp.int32),
            pltpu.VMEM((IBLK, 128), jnp.int32),
            pltpu.VMEM((128, 128), F32),
            pltpu.VMEM((128, 128), F32),
            pltpu.VMEM_SHARED((NPAD, 128), F32),
            pltpu.SemaphoreType.DMA,
            pltpu.SemaphoreType.DMA,
            pltpu.SemaphoreType.DMA,
            pltpu.SemaphoreType.DMA,
        ],
    )


def _h1y_body(t2_ref, deg2_ref, w1_ref, b1_ref, w2_ref, y_ref):
    t = t2_ref[0] + t2_ref[1]
    nd = lax.rsqrt(jnp.maximum(deg2_ref[0], 1.0))
    ns = lax.rsqrt(jnp.maximum(deg2_ref[1], 1.0))
    u = t * nd
    h1 = jnp.maximum(u * w1_ref[...] + b1_ref[...], 0.0)
    h1s = h1 * ns
    y_ref[...] = jnp.dot(h1s, w2_ref[...], preferred_element_type=F32)


def _tc_h1y(t2, deg2, w1, b1, w2):
    nr = NPAD // 1024
    return pl.pallas_call(
        _h1y_body,
        grid=(nr, 4),
        in_specs=[
            pl.BlockSpec((2, 1024, 1), lambda r, p: (0, r, 0)),
            pl.BlockSpec((2, 1024, 1), lambda r, p: (0, r, 0)),
            pl.BlockSpec((1, HID), lambda r, p: (0, 0)),
            pl.BlockSpec((1, HID), lambda r, p: (0, 0)),
            pl.BlockSpec((HID, 128), lambda r, p: (0, p)),
        ],
        out_specs=pl.BlockSpec((1024, 128), lambda r, p: (p * nr + r, 0)),
        out_shape=jax.ShapeDtypeStruct((4 * NPAD, 128), F32),
    )(t2, deg2, w1, b1, w2)


def _pool_mlp_body(agg_ref, din_ref, b2_ref, gid_ref,
                   w1, b1, w2, b2, w3, b3, w4, b4, w5, b5,
                   out_ref, pooled_ref, cnt_ref):
    r = pl.program_id(0)
    nr = pl.num_programs(0)
    agg = jnp.concatenate([agg_ref[i] for i in range(4)], axis=1)
    nd = lax.rsqrt(jnp.maximum(din_ref[...], 1.0))
    h2 = jnp.maximum(agg * nd + b2_ref[...], 0.0)
    gid = gid_ref[...]
    pt = (lax.broadcasted_iota(jnp.int32, (G, 1024), 0)
          == gid).astype(F32)
    pp = jnp.dot(pt, h2, preferred_element_type=F32)
    pc = jnp.dot(pt, jnp.ones((1024, 1), F32), preferred_element_type=F32)

    @pl.when(r == 0)
    def _():
        pooled_ref[...] = pp
        cnt_ref[...] = pc

    @pl.when(r > 0)
    def _():
        pooled_ref[...] += pp
        cnt_ref[...] += pc

    @pl.when(r == nr - 1)
    def _():
        hg = pooled_ref[...] / jnp.maximum(cnt_ref[...], 1.0)
        x = jnp.maximum(jnp.dot(hg, w1[...], preferred_element_type=F32) + b1[...], 0.0)
        x = jnp.maximum(jnp.dot(x, w2[...], preferred_element_type=F32) + b2[...], 0.0)
        x = jnp.maximum(jnp.dot(x, w3[...], preferred_element_type=F32) + b3[...], 0.0)
        x = jnp.maximum(jnp.dot(x, w4[...], preferred_element_type=F32) + b4[...], 0.0)
        logits = jnp.dot(x, w5[...], preferred_element_type=F32) + b5[...]
        m = jnp.max(logits, axis=-1, keepdims=True)
        e = jnp.exp(logits - m)
        out_ref[...] = e / jnp.sum(e, axis=-1, keepdims=True)


def _tc_pool_mlp(agg4, din, b2, gid_row,
                 wc1, bc1, wc2, bc2, wc3, bc3, wc4, bc4, wc5, bc5):
    nr = NPAD // 1024
    full = lambda arr: pl.BlockSpec(arr.shape, lambda r: (0,) * arr.ndim)
    return pl.pallas_call(
        _pool_mlp_body,
        grid=(nr,),
        in_specs=[
            pl.BlockSpec((4, 1024, 128), lambda r: (0, r, 0)),
            pl.BlockSpec((1024, 1), lambda r: (r, 0)),
            pl.BlockSpec((1, HID), lambda r: (0, 0)),
            pl.BlockSpec((1, 1024), lambda r: (0, r)),
            full(wc1), full(bc1), full(wc2), full(bc2), full(wc3),
            full(bc3), full(wc4), full(bc4), full(wc5), full(bc5),
        ],
        out_specs=pl.BlockSpec((G, 10), lambda r: (0, 0)),
        out_shape=jax.ShapeDtypeStruct((G, 10), F32),
        scratch_shapes=[
            pltpu.VMEM((G, HID), F32),
            pltpu.VMEM((G, 1), F32),
        ],
    )(agg4, din, b2, gid_row,
      wc1, bc1, wc2, bc2, wc3, bc3, wc4, bc4, wc5, bc5)


def kernel(edge_index, graph_ids, W1, b1, W2, b2,
           Wc1, bc1, Wc2, bc2, Wc3, bc3, Wc4, bc4, Wc5, bc5):
    E = edge_index.shape[1]
    src = edge_index[0]
    dst = edge_index[1]
    pad = N + (jnp.arange(EPAD - E, dtype=jnp.int32) % (NPAD - N))
    srcp = jnp.concatenate([src, pad]).reshape(ROWS, 128)
    dstp = jnp.concatenate([dst, pad]).reshape(ROWS, 128)
    srcoff = (srcp[None] + (jnp.arange(4, dtype=jnp.int32) * NPAD)[:, None, None]
              ).reshape(4, NSUB, NCHUNK, ACHUNK, 128)
    dstb = jnp.broadcast_to(dstp.reshape(1, NSUB, NCHUNK, ACHUNK, 128),
                            (4, NSUB, NCHUNK, ACHUNK, 128))
    edg = jnp.concatenate([srcoff, dstb], axis=3).reshape(4 * ROWS * 2, 128)

    deg = _deg_kernel()(srcp, dstp)
    deg2 = deg.reshape(2, NPAD, 1)

    t2 = _tsum_kernel()(deg, srcp, dstp).reshape(2, NPAD, 1)

    y = _tc_h1y(t2, deg2, W1, b1.reshape(1, HID), W2)

    agg = _agg_kernel()(y, edg).reshape(4, NPAD, 128)

    gid_row = jnp.concatenate(
        [graph_ids, jnp.full((NPAD - N,), 1000, jnp.int32)]).reshape(1, NPAD)
    return _tc_pool_mlp(agg, deg2[0], b2.reshape(1, HID), gid_row,
                        Wc1, bc1.reshape(1, -1), Wc2, bc2.reshape(1, -1),
                        Wc3, bc3.reshape(1, -1), Wc4, bc4.reshape(1, -1),
                        Wc5, bc5.reshape(1, -1))
